# baseline (speedup 1.0000x reference)
import functools

import numpy as np
import jax
import jax.numpy as jnp
from jax import lax
from jax.experimental import pallas as pl
from jax.experimental.pallas import tpu as pltpu
from jax.experimental.pallas import tpu_sc as plsc

B = 4
L = 3136
D = 192
U = 450
UP = 512
LP = 3200
NW = 32
RPW = (B * L) // NW
SCALE = 1.0 / np.sqrt(192.0)


def _proj_body(x1_ref, x2_ref, w_ref, b_ref, out_ref):
    w = w_ref[...]
    acc = jnp.dot(w[:, :96], x1_ref[0], preferred_element_type=jnp.float32)
    acc = acc + jnp.dot(w[:, 96:], x2_ref[0], preferred_element_type=jnp.float32)
    out_ref[0] = acc + b_ref[...]


def _tc_proj(x1, x2, w, bias):
    return pl.pallas_call(
        _proj_body,
        grid=(B,),
        in_specs=[
            pl.BlockSpec((1, 96, L), lambda b: (b, 0, 0)),
            pl.BlockSpec((1, 96, L), lambda b: (b, 0, 0)),
            pl.BlockSpec((576, D), lambda b: (0, 0)),
            pl.BlockSpec((576, 1), lambda b: (0, 0)),
        ],
        out_specs=pl.BlockSpec((1, 576, L), lambda b: (b, 0, 0)),
        out_shape=jax.ShapeDtypeStruct((B, 576, L), jnp.float32),
    )(x1, x2, w, bias)


def _scores_body(ks_ref, q_ref, out_ref):
    ks = ks_ref[0]
    q = q_ref[0]
    qks = lax.dot_general(ks, q, (((1,), (1,)), ((), ())),
                          preferred_element_type=jnp.float32)
    rvalid = lax.broadcasted_iota(jnp.int32, (UP, 1), 0) < U
    mx = jnp.max(jnp.where(rvalid, qks, -jnp.inf), axis=0, keepdims=True)
    sm = jnp.sum(jnp.where(rvalid, qks, 0.0), axis=0, keepdims=True)
    m = mx - sm * jnp.float32(1.0 / L)

    bits = lax.bitcast_convert_type(m, jnp.int32)
    key = bits ^ (lax.shift_right_arithmetic(bits, 31) & jnp.int32(0x7FFFFFFF))

    cnt0 = jnp.sum((key >= 0).astype(jnp.int32))
    lo0 = jnp.where(cnt0 >= U, jnp.int32(0), jnp.int32(-(2 ** 31)))
    hi0 = jnp.where(cnt0 >= U, jnp.int32(2 ** 31 - 1), jnp.int32(-1))

    def step(_, lh):
        lo, hi = lh
        d = hi - lo
        mid = lo + (lax.shift_right_logical(d, 1) + (d & 1))
        good = jnp.sum((key >= mid).astype(jnp.int32)) >= U
        return jnp.where(good, mid, lo), jnp.where(good, hi, mid - 1)

    kth, _ = lax.fori_loop(0, 31, step, (lo0, hi0))

    c_gt = jnp.sum((key > kth).astype(jnp.int32))
    need = U - c_gt
    li = lax.broadcasted_iota(jnp.int32, (1, L), 1)
    eq = key == kth

    def step2(_, lh):
        lo2, hi2 = lh
        mid = lax.shift_right_logical(lo2 + hi2, 1)
        good = jnp.sum((eq & (li <= mid)).astype(jnp.int32)) >= need
        return jnp.where(good, lo2, mid + 1), jnp.where(good, mid, hi2)

    jt, _ = lax.fori_loop(0, 12, step2, (jnp.int32(0), jnp.int32(L - 1)))

    tbits = kth ^ (lax.shift_right_arithmetic(kth, 31) & jnp.int32(0x7FFFFFFF))
    t = lax.bitcast_convert_type(tbits, jnp.float32)
    out_ref[0] = jnp.concatenate(
        [m,
         jnp.full((1, 16), t, jnp.float32),
         jnp.full((1, 16), jt.astype(jnp.float32), jnp.float32),
         jnp.zeros((1, 32), jnp.float32)], axis=1)


def _tc_scores(ksamp, q):
    return pl.pallas_call(
        _scores_body,
        grid=(B,),
        in_specs=[
            pl.BlockSpec((1, UP, D), lambda b: (b, 0, 0)),
            pl.BlockSpec((1, L, D), lambda b: (b, 0, 0)),
        ],
        out_specs=pl.BlockSpec((1, 1, LP), lambda b: (b, 0, 0)),
        out_shape=jax.ShapeDtypeStruct((B, 1, LP), jnp.float32),
    )(ksamp, q)


def _attn_body(qr_ref, k_ref, v_ref, out_ref):
    qr = qr_ref[0]
    k = k_ref[0]
    v = v_ref[0]
    s = lax.dot_general(qr, k, (((1,), (1,)), ((), ())),
                        preferred_element_type=jnp.float32) * jnp.float32(SCALE)
    mx = jnp.max(s, axis=1, keepdims=True)
    e = jnp.exp(s - mx)
    attn = e / jnp.sum(e, axis=1, keepdims=True)
    upd = jnp.dot(attn, v, preferred_element_type=jnp.float32)
    vmean = jnp.mean(v, axis=0, keepdims=True)
    ri = lax.broadcasted_iota(jnp.int32, (UP, 1), 0)
    out_ref[0] = jnp.where(ri == U, vmean, upd)


def _tc_attn(qred, k, v):
    return pl.pallas_call(
        _attn_body,
        grid=(B,),
        in_specs=[
            pl.BlockSpec((1, UP, D), lambda b: (b, 0, 0)),
            pl.BlockSpec((1, L, D), lambda b: (b, 0, 0)),
            pl.BlockSpec((1, L, D), lambda b: (b, 0, 0)),
        ],
        out_specs=pl.BlockSpec((1, UP, D), lambda b: (b, 0, 0)),
        out_shape=jax.ShapeDtypeStruct((B, UP, D), jnp.float32),
    )(qred, k, v)


def _sc_ksample_body(kflat_hbm, idx_hbm, out_hbm, idx_v, rows_v, sem):
    wid = lax.axis_index("c") * 16 + lax.axis_index("s")
    base = wid * 64
    pltpu.sync_copy(idx_hbm.at[pl.ds(base, 64)], idx_v)
    pltpu.async_copy(kflat_hbm.at[idx_v], rows_v, sem).wait()
    pltpu.sync_copy(rows_v, out_hbm.at[pl.ds(base, 64)])


def _sc_select_body(maux_hbm, qflat_hbm, rsrc_hbm, qred_hbm, m_v, rs_v, il_v, qr_v, sem):
    wid = lax.axis_index("c") * 16 + lax.axis_index("s")

    @pl.when(wid < B)
    def _():
        b = wid
        pltpu.sync_copy(maux_hbm.at[b], m_v)
        zeros16 = jnp.zeros((16,), jnp.int32)
        for j in range(4):
            for i in range(8):
                il_v[j, pl.ds(i * 16, 16)] = zeros16
        tvec = m_v[pl.ds(L, 16)]
        jtvec = m_v[pl.ds(L + 16, 16)].astype(jnp.int32)
        lanes = lax.iota(jnp.int32, 16)

        def cstep(i, cnt):
            v = m_v[pl.ds(i * 16, 16)]
            gidx = lanes + i * 16
            mask = (v > tvec) | ((v == tvec) & (gidx <= jtvec))
            mi = mask.astype(jnp.int32)
            pos = cnt + plsc.cumsum(mi) - 1
            rs_v[pl.ds(i * 16, 16)] = jnp.where(mask, pos + b * UP,
                                                jnp.int32(U) + b * UP)
            plsc.store_scatter(il_v, [pos // 128, pos % 128],
                               gidx + b * L, mask=mask)
            return cnt + plsc.all_reduce_population_count(mask)

        lax.fori_loop(0, L // 16, cstep, jnp.zeros((16,), jnp.int32))

        for j in range(4):
            pltpu.async_copy(qflat_hbm.at[il_v.at[j]],
                             qr_v.at[pl.ds(j * 128, 128)], sem).wait()
        pltpu.sync_copy(qr_v, qred_hbm.at[b])
        pltpu.sync_copy(rs_v, rsrc_hbm.at[pl.ds(b * L, L)])


def _sc_assemble_body(upd_hbm, rsrc_hbm, out_hbm, idx_v, rows_v, sem):
    wid = lax.axis_index("c") * 16 + lax.axis_index("s")
    base = wid * RPW

    def jstep(j, carry):
        pltpu.sync_copy(rsrc_hbm.at[pl.ds(base + j * 56, 56)], idx_v)
        pltpu.async_copy(upd_hbm.at[idx_v], rows_v, sem).wait()
        pltpu.sync_copy(rows_v, out_hbm.at[pl.ds(base + j * 56, 56)])
        return carry

    lax.fori_loop(0, RPW // 56, jstep, jnp.int32(0))


@functools.cache
def _sc_kernels():
    mesh = plsc.VectorSubcoreMesh(core_axis_name="c", subcore_axis_name="s")
    params = pltpu.CompilerParams(use_tc_tiling_on_sc=False,
                                  needs_layout_passes=False)
    ksample = functools.partial(
        pl.kernel,
        compiler_params=params,
        out_type=jax.ShapeDtypeStruct((B * UP, D), jnp.float32),
        mesh=mesh,
        scratch_types=[pltpu.VMEM((64,), jnp.int32),
                       pltpu.VMEM((64, D), jnp.float32),
                       pltpu.SemaphoreType.DMA],
    )(_sc_ksample_body)
    select = functools.partial(
        pl.kernel,
        compiler_params=params,
        out_type=[jax.ShapeDtypeStruct((B * L,), jnp.int32),
                  jax.ShapeDtypeStruct((B, UP, D), jnp.float32)],
        mesh=mesh,
        scratch_types=[pltpu.VMEM((LP,), jnp.float32),
                       pltpu.VMEM((L,), jnp.int32),
                       pltpu.VMEM((4, 128), jnp.int32),
                       pltpu.VMEM((UP, D), jnp.float32),
                       pltpu.SemaphoreType.DMA],
    )(_sc_select_body)
    assemble = functools.partial(
        pl.kernel,
        compiler_params=params,
        out_type=jax.ShapeDtypeStruct((B * L, D), jnp.float32),
        mesh=mesh,
        scratch_types=[pltpu.VMEM((56,), jnp.int32),
                       pltpu.VMEM((56, D), jnp.float32),
                       pltpu.SemaphoreType.DMA],
    )(_sc_assemble_body)
    return ksample, select, assemble


def _sc_ksample(kflat, idxpad):
    return _sc_kernels()[0](kflat, idxpad)


def _sc_select(maux, qflat):
    return _sc_kernels()[1](maux, qflat)


def _sc_assemble(updflat, rsrc):
    return _sc_kernels()[2](updflat, rsrc)


def kernel(input_1, input_2, Wq, bq, Wk, bk, Wv, bv):
    x1 = input_1.reshape(B, 96, L)
    x2 = input_2.reshape(B, 96, L)
    w = jnp.concatenate([Wq, Wk, Wv], axis=0)
    bias = jnp.concatenate([bq, bk, bv], axis=0).reshape(576, 1)

    proj = _tc_proj(x1, x2, w, bias)
    q = proj[:, 0:192, :].reshape(B, L, D)
    k = proj[:, 192:384, :].reshape(B, L, D)
    v = proj[:, 384:576, :].reshape(B, L, D)
    qflat = q.reshape(B * L, D)
    kflat = k.reshape(B * L, D)

    idx_s = jax.random.randint(jax.random.key(42), (U,), 0, L)
    offs = (jnp.arange(B, dtype=jnp.int32) * L)[:, None]
    idxpad = (jnp.zeros((B, UP), jnp.int32)
              .at[:, :U].set(idx_s.astype(jnp.int32)[None, :] + offs)
              .reshape(B * UP))

    ksamp = _sc_ksample(kflat, idxpad)
    maux = _tc_scores(ksamp.reshape(B, UP, D), q)
    rsrc, qred = _sc_select(maux.reshape(B, LP), qflat)
    upd = _tc_attn(qred, k, v)
    outf = _sc_assemble(upd.reshape(B * UP, D), rsrc)
    return outf.reshape(B, D, 56, 56)


# speedup vs baseline: 1.2525x; 1.2525x over previous
import functools

import numpy as np
import jax
import jax.numpy as jnp
from jax import lax
from jax.experimental import pallas as pl
from jax.experimental.pallas import tpu as pltpu
from jax.experimental.pallas import tpu_sc as plsc

B = 4
L = 3136
D = 192
U = 450
UP = 512
LP = 3200
NW = 32
RPW = (B * L) // NW
SCALE = 1.0 / np.sqrt(192.0)


def _proj_body(x1_ref, x2_ref, w_ref, b_ref, out_ref):
    w = w_ref[...]
    acc = jnp.dot(w[:, :96], x1_ref[0], preferred_element_type=jnp.float32)
    acc = acc + jnp.dot(w[:, 96:], x2_ref[0], preferred_element_type=jnp.float32)
    out_ref[0] = acc + b_ref[...]


def _tc_proj(x1, x2, w, bias):
    return pl.pallas_call(
        _proj_body,
        grid=(B,),
        in_specs=[
            pl.BlockSpec((1, 96, L), lambda b: (b, 0, 0)),
            pl.BlockSpec((1, 96, L), lambda b: (b, 0, 0)),
            pl.BlockSpec((576, D), lambda b: (0, 0)),
            pl.BlockSpec((576, 1), lambda b: (0, 0)),
        ],
        out_specs=pl.BlockSpec((1, 576, L), lambda b: (b, 0, 0)),
        out_shape=jax.ShapeDtypeStruct((B, 576, L), jnp.float32),
    )(x1, x2, w, bias)


def _scores_body(ks_ref, q_ref, out_ref):
    ks = ks_ref[0]
    q = q_ref[0]
    qks = lax.dot_general(ks, q, (((1,), (1,)), ((), ())),
                          preferred_element_type=jnp.float32)
    rvalid = lax.broadcasted_iota(jnp.int32, (UP, 1), 0) < U
    mx = jnp.max(jnp.where(rvalid, qks, -jnp.inf), axis=0, keepdims=True)
    sm = jnp.sum(jnp.where(rvalid, qks, 0.0), axis=0, keepdims=True)
    m = mx - sm * jnp.float32(1.0 / L)

    bits = lax.bitcast_convert_type(m, jnp.int32)
    key = bits ^ (lax.shift_right_arithmetic(bits, 31) & jnp.int32(0x7FFFFFFF))

    cnt0 = jnp.sum((key >= 0).astype(jnp.int32))
    lo0 = jnp.where(cnt0 >= U, jnp.int32(0), jnp.int32(-(2 ** 31)))
    hi0 = jnp.where(cnt0 >= U, jnp.int32(2 ** 31 - 1), jnp.int32(-1))

    def step(_, lh):
        lo, hi = lh
        d = hi - lo
        mid = lo + (lax.shift_right_logical(d, 1) + (d & 1))
        good = jnp.sum((key >= mid).astype(jnp.int32)) >= U
        return jnp.where(good, mid, lo), jnp.where(good, hi, mid - 1)

    kth, _ = lax.fori_loop(0, 31, step, (lo0, hi0))

    c_gt = jnp.sum((key > kth).astype(jnp.int32))
    need = U - c_gt
    li = lax.broadcasted_iota(jnp.int32, (1, L), 1)
    eq = key == kth

    def step2(_, lh):
        lo2, hi2 = lh
        mid = lax.shift_right_logical(lo2 + hi2, 1)
        good = jnp.sum((eq & (li <= mid)).astype(jnp.int32)) >= need
        return jnp.where(good, lo2, mid + 1), jnp.where(good, mid, hi2)

    jt, _ = lax.fori_loop(0, 12, step2, (jnp.int32(0), jnp.int32(L - 1)))

    tbits = kth ^ (lax.shift_right_arithmetic(kth, 31) & jnp.int32(0x7FFFFFFF))
    t = lax.bitcast_convert_type(tbits, jnp.float32)
    out_ref[0] = jnp.concatenate(
        [m,
         jnp.full((1, 16), t, jnp.float32),
         jnp.full((1, 16), jt.astype(jnp.float32), jnp.float32),
         jnp.zeros((1, 32), jnp.float32)], axis=1)


def _tc_scores(ksamp, q):
    return pl.pallas_call(
        _scores_body,
        grid=(B,),
        in_specs=[
            pl.BlockSpec((1, UP, D), lambda b: (b, 0, 0)),
            pl.BlockSpec((1, L, D), lambda b: (b, 0, 0)),
        ],
        out_specs=pl.BlockSpec((1, 1, LP), lambda b: (b, 0, 0)),
        out_shape=jax.ShapeDtypeStruct((B, 1, LP), jnp.float32),
    )(ksamp, q)


def _attn_body(qr_ref, k_ref, v_ref, out_ref):
    qr = qr_ref[0]
    k = k_ref[0]
    v = v_ref[0]
    s = lax.dot_general(qr, k, (((1,), (1,)), ((), ())),
                        preferred_element_type=jnp.float32) * jnp.float32(SCALE)
    mx = jnp.max(s, axis=1, keepdims=True)
    e = jnp.exp(s - mx)
    attn = e / jnp.sum(e, axis=1, keepdims=True)
    upd = jnp.dot(attn, v, preferred_element_type=jnp.float32)
    vmean = jnp.mean(v, axis=0, keepdims=True)
    ri = lax.broadcasted_iota(jnp.int32, (UP, 1), 0)
    out_ref[0] = jnp.where(ri == U, vmean, upd)


def _tc_attn(qred, k, v):
    return pl.pallas_call(
        _attn_body,
        grid=(B,),
        in_specs=[
            pl.BlockSpec((1, UP, D), lambda b: (b, 0, 0)),
            pl.BlockSpec((1, L, D), lambda b: (b, 0, 0)),
            pl.BlockSpec((1, L, D), lambda b: (b, 0, 0)),
        ],
        out_specs=pl.BlockSpec((1, UP, D), lambda b: (b, 0, 0)),
        out_shape=jax.ShapeDtypeStruct((B, UP, D), jnp.float32),
    )(qred, k, v)


def _sc_ksample_body(kflat_hbm, idx_hbm, out_hbm, idx_v, rows_v, sem):
    wid = lax.axis_index("c") * 16 + lax.axis_index("s")
    base = wid * 64
    pltpu.sync_copy(idx_hbm.at[pl.ds(base, 64)], idx_v)
    pltpu.async_copy(kflat_hbm.at[idx_v], rows_v, sem).wait()
    pltpu.sync_copy(rows_v, out_hbm.at[pl.ds(base, 64)])


def _sc_select_body(maux_hbm, qflat_hbm, dlist_hbm, qred_hbm, m_v, il_v, qr_v, sem):
    cid = lax.axis_index("c")
    sid = lax.axis_index("s")

    @pl.when(sid < 2)
    def _():
        b = cid * 2 + sid
        pltpu.sync_copy(maux_hbm.at[b], m_v)
        zeros16 = jnp.zeros((16,), jnp.int32)
        for j in range(4):
            for i in range(8):
                il_v[j, pl.ds(i * 16, 16)] = zeros16
        tvec = m_v[pl.ds(L, 16)]
        jtvec = m_v[pl.ds(L + 16, 16)].astype(jnp.int32)
        lanes = lax.iota(jnp.int32, 16)

        def cstep(i, cnt):
            v = m_v[pl.ds(i * 16, 16)]
            gidx = lanes + i * 16
            mask = (v > tvec) | ((v == tvec) & (gidx <= jtvec))
            mi = mask.astype(jnp.int32)
            pos = cnt + plsc.cumsum(mi) - 1
            plsc.store_scatter(il_v, [pos // 128, pos % 128],
                               gidx + b * L, mask=mask)
            return cnt + plsc.all_reduce_population_count(mask)

        lax.fori_loop(0, L // 16, cstep, jnp.zeros((16,), jnp.int32))

        for j in range(4):
            pltpu.async_copy(qflat_hbm.at[il_v.at[j]],
                             qr_v.at[pl.ds(j * 128, 128)], sem).wait()
        pltpu.sync_copy(qr_v, qred_hbm.at[b])
        lanes = lax.iota(jnp.int32, 16)
        for i in range(4, 8):
            cur = il_v[3, pl.ds(i * 16, 16)]
            pad = (lanes + i * 16) >= (U - 384)
            il_v[3, pl.ds(i * 16, 16)] = jnp.where(pad, jnp.int32(B * L), cur)
        pltpu.sync_copy(il_v, dlist_hbm.at[b])


def _sc_assemble_body(upd_hbm, dlist_hbm, out_hbm, vrow_v, fill_v, idx_v,
                      rows_v, sem, wsem):
    cid = lax.axis_index("c")
    sid = lax.axis_index("s")
    wid = cid * 16 + sid
    base = wid * RPW
    b = wid // 8
    pltpu.sync_copy(upd_hbm.at[b * UP + U], vrow_v)

    def rstep(r, carry):
        for j in range(D // 16):
            fill_v[r, pl.ds(j * 16, 16)] = vrow_v[pl.ds(j * 16, 16)]
        return carry

    lax.fori_loop(0, 56, rstep, jnp.int32(0))
    for t in range(RPW // 56):
        pltpu.async_copy(fill_v, out_hbm.at[pl.ds(base + t * 56, 56)], wsem)
    for t in range(RPW // 56):
        pltpu.make_async_copy(fill_v, out_hbm.at[pl.ds(base + t * 56, 56)],
                              wsem).wait()
    plsc.subcore_barrier()
    g0 = cid * (2 * UP) + sid * 64
    pltpu.sync_copy(dlist_hbm.at[pl.ds(g0, 64)], idx_v)
    pltpu.sync_copy(upd_hbm.at[pl.ds(g0, 64)], rows_v)
    pltpu.async_copy(rows_v, out_hbm.at[idx_v], sem).wait()


@functools.cache
def _sc_kernels():
    mesh = plsc.VectorSubcoreMesh(core_axis_name="c", subcore_axis_name="s")
    params = pltpu.CompilerParams(use_tc_tiling_on_sc=False,
                                  needs_layout_passes=False)
    ksample = functools.partial(
        pl.kernel,
        compiler_params=params,
        out_type=jax.ShapeDtypeStruct((B * UP, D), jnp.float32),
        mesh=mesh,
        scratch_types=[pltpu.VMEM((64,), jnp.int32),
                       pltpu.VMEM((64, D), jnp.float32),
                       pltpu.SemaphoreType.DMA],
    )(_sc_ksample_body)
    select = functools.partial(
        pl.kernel,
        compiler_params=params,
        out_type=[jax.ShapeDtypeStruct((B, 4, 128), jnp.int32),
                  jax.ShapeDtypeStruct((B, UP, D), jnp.float32)],
        mesh=mesh,
        scratch_types=[pltpu.VMEM((LP,), jnp.float32),
                       pltpu.VMEM((4, 128), jnp.int32),
                       pltpu.VMEM((UP, D), jnp.float32),
                       pltpu.SemaphoreType.DMA],
    )(_sc_select_body)
    assemble = functools.partial(
        pl.kernel,
        compiler_params=params,
        out_type=jax.ShapeDtypeStruct((B * L + 8, D), jnp.float32),
        mesh=mesh,
        scratch_types=[pltpu.VMEM((D,), jnp.float32),
                       pltpu.VMEM((56, D), jnp.float32),
                       pltpu.VMEM((64,), jnp.int32),
                       pltpu.VMEM((64, D), jnp.float32),
                       pltpu.SemaphoreType.DMA,
                       pltpu.SemaphoreType.DMA],
    )(_sc_assemble_body)
    return ksample, select, assemble


def _sc_ksample(kflat, idxpad):
    return _sc_kernels()[0](kflat, idxpad)


def _sc_select(maux, qflat):
    return _sc_kernels()[1](maux, qflat)


def _sc_assemble(updflat, dlist):
    return _sc_kernels()[2](updflat, dlist)


def kernel(input_1, input_2, Wq, bq, Wk, bk, Wv, bv):
    x1 = input_1.reshape(B, 96, L)
    x2 = input_2.reshape(B, 96, L)
    w = jnp.concatenate([Wq, Wk, Wv], axis=0)
    bias = jnp.concatenate([bq, bk, bv], axis=0).reshape(576, 1)

    proj = _tc_proj(x1, x2, w, bias)
    q = proj[:, 0:192, :].reshape(B, L, D)
    k = proj[:, 192:384, :].reshape(B, L, D)
    v = proj[:, 384:576, :].reshape(B, L, D)
    qflat = q.reshape(B * L, D)
    kflat = k.reshape(B * L, D)

    idx_s = jax.random.randint(jax.random.key(42), (U,), 0, L)
    offs = (jnp.arange(B, dtype=jnp.int32) * L)[:, None]
    idxpad = (jnp.zeros((B, UP), jnp.int32)
              .at[:, :U].set(idx_s.astype(jnp.int32)[None, :] + offs)
              .reshape(B * UP))

    ksamp = _sc_ksample(kflat, idxpad)
    maux = _tc_scores(ksamp.reshape(B, UP, D), q)
    dlist, qred = _sc_select(maux.reshape(B, LP), qflat)
    upd = _tc_attn(qred, k, v)
    outf = _sc_assemble(upd.reshape(B * UP, D),
                        dlist.reshape(B * UP))
    return outf[:B * L].reshape(B, D, 56, 56)


# speedup vs baseline: 1.3259x; 1.0586x over previous
import functools

import numpy as np
import jax
import jax.numpy as jnp
from jax import lax
from jax.experimental import pallas as pl
from jax.experimental.pallas import tpu as pltpu
from jax.experimental.pallas import tpu_sc as plsc

B = 4
L = 3136
D = 192
U = 450
UP = 512
LP = 3200
NW = 32
RPW = (B * L) // NW
SCALE = 1.0 / np.sqrt(192.0)


def _proj_body(x1_ref, x2_ref, w_ref, b_ref, q_ref, k_ref, v_ref):
    w = w_ref[...]
    acc = jnp.dot(w[:, :96], x1_ref[0], preferred_element_type=jnp.float32)
    acc = acc + jnp.dot(w[:, 96:], x2_ref[0], preferred_element_type=jnp.float32)
    acc = acc + b_ref[...]
    q_ref[0] = acc[0:D]
    k_ref[0] = acc[D:2 * D]
    v_ref[0] = acc[2 * D:3 * D]


def _tc_proj(x1, x2, w, bias):
    spec = pl.BlockSpec((1, D, L), lambda b: (b, 0, 0))
    return pl.pallas_call(
        _proj_body,
        grid=(B,),
        in_specs=[
            pl.BlockSpec((1, 96, L), lambda b: (b, 0, 0)),
            pl.BlockSpec((1, 96, L), lambda b: (b, 0, 0)),
            pl.BlockSpec((576, D), lambda b: (0, 0)),
            pl.BlockSpec((576, 1), lambda b: (0, 0)),
        ],
        out_specs=[spec, spec, spec],
        out_shape=[jax.ShapeDtypeStruct((B, D, L), jnp.float32)] * 3,
    )(x1, x2, w, bias)


def _scores_body(ks_ref, q_ref, out_ref):
    ks = ks_ref[0]
    q = q_ref[0]
    qks = lax.dot_general(ks, q, (((1,), (1,)), ((), ())),
                          preferred_element_type=jnp.float32)
    rvalid = lax.broadcasted_iota(jnp.int32, (UP, 1), 0) < U
    mx = jnp.max(jnp.where(rvalid, qks, -jnp.inf), axis=0, keepdims=True)
    sm = jnp.sum(jnp.where(rvalid, qks, 0.0), axis=0, keepdims=True)
    m = mx - sm * jnp.float32(1.0 / L)

    bits = lax.bitcast_convert_type(m, jnp.int32)
    key = bits ^ (lax.shift_right_arithmetic(bits, 31) & jnp.int32(0x7FFFFFFF))

    cnt0 = jnp.sum((key >= 0).astype(jnp.int32))
    lo0 = jnp.where(cnt0 >= U, jnp.int32(0), jnp.int32(-(2 ** 31)))
    hi0 = jnp.where(cnt0 >= U, jnp.int32(2 ** 31 - 1), jnp.int32(-1))

    def step(_, lh):
        lo, hi = lh
        d = hi - lo
        mid = lo + (lax.shift_right_logical(d, 1) + (d & 1))
        good = jnp.sum((key >= mid).astype(jnp.int32)) >= U
        return jnp.where(good, mid, lo), jnp.where(good, hi, mid - 1)

    kth, _ = lax.fori_loop(0, 31, step, (lo0, hi0))

    c_gt = jnp.sum((key > kth).astype(jnp.int32))
    need = U - c_gt
    li = lax.broadcasted_iota(jnp.int32, (1, L), 1)
    eq = key == kth

    def step2(_, lh):
        lo2, hi2 = lh
        mid = lax.shift_right_logical(lo2 + hi2, 1)
        good = jnp.sum((eq & (li <= mid)).astype(jnp.int32)) >= need
        return jnp.where(good, lo2, mid + 1), jnp.where(good, mid, hi2)

    jt, _ = lax.fori_loop(0, 12, step2, (jnp.int32(0), jnp.int32(L - 1)))

    tbits = kth ^ (lax.shift_right_arithmetic(kth, 31) & jnp.int32(0x7FFFFFFF))
    t = lax.bitcast_convert_type(tbits, jnp.float32)
    out_ref[0] = jnp.concatenate(
        [m,
         jnp.full((1, 16), t, jnp.float32),
         jnp.full((1, 16), jt.astype(jnp.float32), jnp.float32),
         jnp.zeros((1, 32), jnp.float32)], axis=1)


def _tc_scores(ksamp, q):
    return pl.pallas_call(
        _scores_body,
        grid=(B,),
        in_specs=[
            pl.BlockSpec((1, UP, D), lambda b: (b, 0, 0)),
            pl.BlockSpec((1, L, D), lambda b: (b, 0, 0)),
        ],
        out_specs=pl.BlockSpec((1, 1, LP), lambda b: (b, 0, 0)),
        out_shape=jax.ShapeDtypeStruct((B, 1, LP), jnp.float32),
    )(ksamp, q)


def _attn_body(qr_ref, k_ref, v_ref, out_ref):
    qr = qr_ref[0]
    k = k_ref[0]
    v = v_ref[0]
    s = lax.dot_general(qr, k, (((1,), (1,)), ((), ())),
                        preferred_element_type=jnp.float32) * jnp.float32(SCALE)
    mx = jnp.max(s, axis=1, keepdims=True)
    e = jnp.exp(s - mx)
    attn = e / jnp.sum(e, axis=1, keepdims=True)
    upd = jnp.dot(attn, v, preferred_element_type=jnp.float32)
    vmean = jnp.mean(v, axis=0, keepdims=True)
    ri = lax.broadcasted_iota(jnp.int32, (UP, 1), 0)
    out_ref[0] = jnp.where(ri == U, vmean, upd)


def _tc_attn(qred, k, v):
    return pl.pallas_call(
        _attn_body,
        grid=(B,),
        in_specs=[
            pl.BlockSpec((1, UP, D), lambda b: (b, 0, 0)),
            pl.BlockSpec((1, L, D), lambda b: (b, 0, 0)),
            pl.BlockSpec((1, L, D), lambda b: (b, 0, 0)),
        ],
        out_specs=pl.BlockSpec((1, UP, D), lambda b: (b, 0, 0)),
        out_shape=jax.ShapeDtypeStruct((B, UP, D), jnp.float32),
    )(qred, k, v)


def _sc_ksample_body(kflat_hbm, idx_hbm, out_hbm, idx_v, rows_v, sem):
    wid = lax.axis_index("c") * 16 + lax.axis_index("s")
    base = wid * 64
    pltpu.sync_copy(idx_hbm.at[pl.ds(base, 64)], idx_v)
    pltpu.async_copy(kflat_hbm.at[idx_v], rows_v, sem).wait()
    pltpu.sync_copy(rows_v, out_hbm.at[pl.ds(base, 64)])


def _sc_select_body(maux_hbm, qflat_hbm, dlist_hbm, qred_hbm, m_v, il_v, qr_v, sem):
    cid = lax.axis_index("c")
    sid = lax.axis_index("s")

    @pl.when(sid < 2)
    def _():
        b = cid * 2 + sid
        pltpu.sync_copy(maux_hbm.at[b], m_v)
        zeros16 = jnp.zeros((16,), jnp.int32)
        for j in range(4):
            for i in range(8):
                il_v[j, pl.ds(i * 16, 16)] = zeros16
        tvec = m_v[pl.ds(L, 16)]
        jtvec = m_v[pl.ds(L + 16, 16)].astype(jnp.int32)
        lanes = lax.iota(jnp.int32, 16)

        def cstep(i, cnt):
            v = m_v[pl.ds(i * 16, 16)]
            gidx = lanes + i * 16
            mask = (v > tvec) | ((v == tvec) & (gidx <= jtvec))
            mi = mask.astype(jnp.int32)
            pos = cnt + plsc.cumsum(mi) - 1
            plsc.store_scatter(il_v, [pos // 128, pos % 128],
                               gidx + b * L, mask=mask)
            return cnt + plsc.all_reduce_population_count(mask)

        lax.fori_loop(0, L // 16, cstep, jnp.zeros((16,), jnp.int32))

        for j in range(4):
            pltpu.async_copy(qflat_hbm.at[il_v.at[j]],
                             qr_v.at[pl.ds(j * 128, 128)], sem).wait()
        pltpu.sync_copy(qr_v, qred_hbm.at[b])
        lanes = lax.iota(jnp.int32, 16)
        for i in range(4, 8):
            cur = il_v[3, pl.ds(i * 16, 16)]
            pad = (lanes + i * 16) >= (U - 384)
            il_v[3, pl.ds(i * 16, 16)] = jnp.where(pad, jnp.int32(B * L), cur)
        pltpu.sync_copy(il_v, dlist_hbm.at[b])


def _sc_assemble_body(upd_hbm, dlist_hbm, out_hbm, vrow_v, fill_v, idx_v,
                      rows_v, sem, wsem):
    cid = lax.axis_index("c")
    sid = lax.axis_index("s")
    wid = cid * 16 + sid
    base = wid * RPW
    b = wid // 8
    pltpu.sync_copy(upd_hbm.at[b * UP + U], vrow_v)

    def rstep(r, carry):
        for j in range(D // 16):
            fill_v[r, pl.ds(j * 16, 16)] = vrow_v[pl.ds(j * 16, 16)]
        return carry

    lax.fori_loop(0, 56, rstep, jnp.int32(0))
    for t in range(RPW // 56):
        pltpu.async_copy(fill_v, out_hbm.at[pl.ds(base + t * 56, 56)], wsem)
    for t in range(RPW // 56):
        pltpu.make_async_copy(fill_v, out_hbm.at[pl.ds(base + t * 56, 56)],
                              wsem).wait()
    plsc.subcore_barrier()
    g0 = cid * (2 * UP) + sid * 64
    pltpu.sync_copy(dlist_hbm.at[pl.ds(g0, 64)], idx_v)
    pltpu.sync_copy(upd_hbm.at[pl.ds(g0, 64)], rows_v)
    pltpu.async_copy(rows_v, out_hbm.at[idx_v], sem).wait()


@functools.cache
def _sc_kernels():
    mesh = plsc.VectorSubcoreMesh(core_axis_name="c", subcore_axis_name="s")
    params = pltpu.CompilerParams(use_tc_tiling_on_sc=False,
                                  needs_layout_passes=False)
    ksample = functools.partial(
        pl.kernel,
        compiler_params=params,
        out_type=jax.ShapeDtypeStruct((B * UP, D), jnp.float32),
        mesh=mesh,
        scratch_types=[pltpu.VMEM((64,), jnp.int32),
                       pltpu.VMEM((64, D), jnp.float32),
                       pltpu.SemaphoreType.DMA],
    )(_sc_ksample_body)
    select = functools.partial(
        pl.kernel,
        compiler_params=params,
        out_type=[jax.ShapeDtypeStruct((B, 4, 128), jnp.int32),
                  jax.ShapeDtypeStruct((B, UP, D), jnp.float32)],
        mesh=mesh,
        scratch_types=[pltpu.VMEM((LP,), jnp.float32),
                       pltpu.VMEM((4, 128), jnp.int32),
                       pltpu.VMEM((UP, D), jnp.float32),
                       pltpu.SemaphoreType.DMA],
    )(_sc_select_body)
    assemble = functools.partial(
        pl.kernel,
        compiler_params=params,
        out_type=jax.ShapeDtypeStruct((B * L + 8, D), jnp.float32),
        mesh=mesh,
        scratch_types=[pltpu.VMEM((D,), jnp.float32),
                       pltpu.VMEM((56, D), jnp.float32),
                       pltpu.VMEM((64,), jnp.int32),
                       pltpu.VMEM((64, D), jnp.float32),
                       pltpu.SemaphoreType.DMA,
                       pltpu.SemaphoreType.DMA],
    )(_sc_assemble_body)
    return ksample, select, assemble


def _sc_ksample(kflat, idxpad):
    return _sc_kernels()[0](kflat, idxpad)


def _sc_select(maux, qflat):
    return _sc_kernels()[1](maux, qflat)


def _sc_assemble(updflat, dlist):
    return _sc_kernels()[2](updflat, dlist)


def kernel(input_1, input_2, Wq, bq, Wk, bk, Wv, bv):
    x1 = input_1.reshape(B, 96, L)
    x2 = input_2.reshape(B, 96, L)
    w = jnp.concatenate([Wq, Wk, Wv], axis=0)
    bias = jnp.concatenate([bq, bk, bv], axis=0).reshape(576, 1)

    q_out, k_out, v_out = _tc_proj(x1, x2, w, bias)
    q = q_out.reshape(B, L, D)
    k = k_out.reshape(B, L, D)
    v = v_out.reshape(B, L, D)
    qflat = q.reshape(B * L, D)
    kflat = k.reshape(B * L, D)

    idx_s = jax.random.randint(jax.random.key(42), (U,), 0, L)
    offs = (jnp.arange(B, dtype=jnp.int32) * L)[:, None]
    idxpad = (jnp.zeros((B, UP), jnp.int32)
              .at[:, :U].set(idx_s.astype(jnp.int32)[None, :] + offs)
              .reshape(B * UP))

    ksamp = _sc_ksample(kflat, idxpad)
    maux = _tc_scores(ksamp.reshape(B, UP, D), q)
    dlist, qred = _sc_select(maux.reshape(B, LP), qflat)
    upd = _tc_attn(qred, k, v)
    outf = _sc_assemble(upd.reshape(B * UP, D),
                        dlist.reshape(B * UP))
    return outf[:B * L].reshape(B, D, 56, 56)
